# Initial kernel scaffold; baseline (speedup 1.0000x reference)
import jax
import jax.numpy as jnp
from jax.experimental import pallas as pl


def kernel(pos, batch, W1a, W1b, W1c, W2a, W2b, W2c, Wf2a, Wf2b, Wf1a, Wf1b, Wr1, Wr2, br2, Wc1, Wc2, bc2):
    raise NotImplementedError("write your pallas kernel here")


# speedup vs baseline: 4.1692x; 4.1692x over previous
import functools

import jax
import jax.numpy as jnp
from jax.experimental import pallas as pl
from jax.experimental.pallas import tpu as pltpu

_B, _P = 8, 1024
_M1, _M2 = 512, 128
_K = 64
_EPS = 1e-5
_NEG = -1e30
_R1SQ = 0.2 * 0.2
_R2SQ = 0.4 * 0.4

_f32 = jnp.float32
_i32 = jnp.int32


def _b16(x):
    return x.astype(jnp.bfloat16).astype(_f32)


def _fps_select(X, Y, Z, n, m):
    iota = jax.lax.broadcasted_iota(_i32, (_B, n), 1)
    iom = jax.lax.broadcasted_iota(_i32, (_B, m), 1)

    def step(i, c):
        dists, last, xa, ya, za = c
        oh = (iota == last).astype(_f32)
        lx = jnp.sum(X * oh, axis=1, keepdims=True)
        ly = jnp.sum(Y * oh, axis=1, keepdims=True)
        lz = jnp.sum(Z * oh, axis=1, keepdims=True)
        sel = iom == i
        xa = jnp.where(sel, lx, xa)
        ya = jnp.where(sel, ly, ya)
        za = jnp.where(sel, lz, za)
        dx = X - lx
        dy = Y - ly
        dz = Z - lz
        d = (dx * dx + dy * dy) + dz * dz
        dists = jnp.minimum(dists, d)
        mx = jnp.max(dists, axis=1, keepdims=True)
        cand = jnp.where(dists == mx, iota, n)
        nxt = jnp.min(cand, axis=1, keepdims=True).astype(_i32)
        return dists, nxt, xa, ya, za

    dists0 = jnp.full((_B, n), jnp.inf, _f32)
    last0 = jnp.zeros((_B, 1), _i32)
    z = jnp.zeros((_B, m), _f32)
    _, _, xa, ya, za = jax.lax.fori_loop(0, m, step, (dists0, last0, z, z, z))
    return xa, ya, za


def _fps_kernel(pos_ref, px1, py1, pz1, px2, py2, pz2):
    X = pos_ref[0]
    Y = pos_ref[1]
    Z = pos_ref[2]
    xa, ya, za = _fps_select(X, Y, Z, _P, _M1)
    px1[...] = xa
    py1[...] = ya
    pz1[...] = za
    xb, yb, zb = _fps_select(xa, ya, za, _M1, _M2)
    px2[...] = xb
    py2[...] = yb
    pz2[...] = zb


def _run_fps(pos_t):
    outs = [jax.ShapeDtypeStruct((_B, _M1), _f32)] * 3 + \
           [jax.ShapeDtypeStruct((_B, _M2), _f32)] * 3
    return pl.pallas_call(_fps_kernel, out_shape=outs)(pos_t)


def _radius_kernel(src_ref, dst_ref, rx_ref, ry_ref, rz_ref, va_ref, ix_ref,
                   d2_ref, *, M, N, r2):
    xs = src_ref[0, 0]
    ys = src_ref[0, 1]
    zs = src_ref[0, 2]
    xd = dst_ref[0, :, 0:1]
    yd = dst_ref[0, :, 1:2]
    zd = dst_ref[0, :, 2:3]
    dx = xd - xs
    dy = yd - ys
    dz = zd - zs
    d2_ref[...] = (dx * dx + dy * dy) + dz * dz
    iota = jax.lax.broadcasted_iota(_i32, (M, N), 1)
    iok = jax.lax.broadcasted_iota(_i32, (M, _K), 1)

    def step(t, c):
        rxa, rya, rza, vaa, ixa = c
        d2 = d2_ref[...]
        mn = jnp.min(d2, axis=1, keepdims=True)
        cand = jnp.where(d2 == mn, iota, N)
        idx = jnp.min(cand, axis=1, keepdims=True).astype(_i32)
        ohm = iota == idx
        oh = ohm.astype(_f32)
        gx = jnp.sum(xs * oh, axis=1, keepdims=True)
        gy = jnp.sum(ys * oh, axis=1, keepdims=True)
        gz = jnp.sum(zs * oh, axis=1, keepdims=True)
        valid = (mn <= r2).astype(_f32)
        selk = iok == t
        rxa = jnp.where(selk, gx - xd, rxa)
        rya = jnp.where(selk, gy - yd, rya)
        rza = jnp.where(selk, gz - zd, rza)
        vaa = jnp.where(selk, valid, vaa)
        ixa = jnp.where(selk, idx, ixa)
        d2_ref[...] = jnp.where(ohm, jnp.inf, d2)
        return rxa, rya, rza, vaa, ixa

    zf = jnp.zeros((M, _K), _f32)
    zi = jnp.zeros((M, _K), _i32)
    rxa, rya, rza, vaa, ixa = jax.lax.fori_loop(
        0, _K, step, (zf, zf, zf, zf, zi))
    rx_ref[0] = rxa
    ry_ref[0] = rya
    rz_ref[0] = rza
    va_ref[0] = vaa
    ix_ref[0] = ixa


def _run_radius(src4, dst_r, M, N, r2):
    body = functools.partial(_radius_kernel, M=M, N=N, r2=r2)
    outs = [jax.ShapeDtypeStruct((_B, M, _K), _f32)] * 4 + \
           [jax.ShapeDtypeStruct((_B, M, _K), _i32)]
    return pl.pallas_call(
        body,
        grid=(_B,),
        in_specs=[
            pl.BlockSpec((1, 3, 1, N), lambda b: (b, 0, 0, 0)),
            pl.BlockSpec((1, M, 3), lambda b: (b, 0, 0)),
        ],
        out_specs=[pl.BlockSpec((1, M, _K), lambda b: (b, 0, 0))] * 5,
        scratch_shapes=[pltpu.VMEM((M, N), _f32)],
        out_shape=outs,
    )(src4, dst_r)


def _l1_kernel(rx_ref, ry_ref, rz_ref, v_ref, w_ref,
               h_ref, ss_ref, sq_ref, dn_ref):
    w = _b16(w_ref[...])
    F = w.shape[1]
    w0 = w[0:1, :].reshape(1, 1, F)
    w1 = w[1:2, :].reshape(1, 1, F)
    w2 = w[2:3, :].reshape(1, 1, F)
    rx = _b16(rx_ref[...])[..., None]
    ry = _b16(ry_ref[...])[..., None]
    rz = _b16(rz_ref[...])[..., None]
    h = rx * w0 + ry * w1 + rz * w2
    h_ref[...] = h
    v = v_ref[...]
    hm = h * v[..., None]

    @pl.when(pl.program_id(0) == 0)
    def _init():
        ss_ref[...] = jnp.zeros_like(ss_ref)
        sq_ref[...] = jnp.zeros_like(sq_ref)
        dn_ref[...] = jnp.zeros_like(dn_ref)

    ss_ref[...] += jnp.sum(jnp.sum(hm, axis=1), axis=0, keepdims=True)
    sq_ref[...] += jnp.sum(jnp.sum(h * hm, axis=1), axis=0, keepdims=True)
    dn_ref[...] += jnp.sum(v).reshape(1, 1)


def _run_l1(rx, ry, rz, v, w, Tc=256):
    Nc = rx.shape[0]
    F = w.shape[1]
    G = Nc // Tc
    outs = [jax.ShapeDtypeStruct((Nc, _K, F), _f32),
            jax.ShapeDtypeStruct((1, F), _f32),
            jax.ShapeDtypeStruct((1, F), _f32),
            jax.ShapeDtypeStruct((1, 1), _f32)]
    pk = pl.BlockSpec((Tc, _K), lambda i: (i, 0))
    return pl.pallas_call(
        _l1_kernel,
        grid=(G,),
        in_specs=[pk, pk, pk, pk, pl.BlockSpec((3, F), lambda i: (0, 0))],
        out_specs=[
            pl.BlockSpec((Tc, _K, F), lambda i: (i, 0, 0)),
            pl.BlockSpec((1, F), lambda i: (0, 0)),
            pl.BlockSpec((1, F), lambda i: (0, 0)),
            pl.BlockSpec((1, 1), lambda i: (0, 0)),
        ],
        out_shape=outs,
    )(rx, ry, rz, v, w)


def _bnmm_kernel(h_ref, ss_ref, sq_ref, dn_ref, v_ref, w_ref,
                 o_ref, oss_ref, osq_ref):
    den = dn_ref[0, 0]
    F = h_ref.shape[2]
    Fo = w_ref.shape[1]
    Tc = h_ref.shape[0]
    mu2 = ss_ref[...] / den
    var = sq_ref[...] / den - mu2 * mu2
    mu = mu2.reshape(1, 1, F)
    sd = jnp.sqrt(var + _EPS).reshape(1, 1, F)
    x = (h_ref[...] - mu) / sd
    x = jnp.maximum(x, 0.0)
    o2 = jnp.dot(x.reshape(Tc * _K, F), w_ref[...],
                 preferred_element_type=_f32)
    o = o2.reshape(Tc, _K, Fo)
    o_ref[...] = o
    om = o * v_ref[...][..., None]

    @pl.when(pl.program_id(0) == 0)
    def _init():
        oss_ref[...] = jnp.zeros_like(oss_ref)
        osq_ref[...] = jnp.zeros_like(osq_ref)

    oss_ref[...] += jnp.sum(jnp.sum(om, axis=1), axis=0, keepdims=True)
    osq_ref[...] += jnp.sum(jnp.sum(o * om, axis=1), axis=0, keepdims=True)


def _run_bnmm(h, ss, sq, dn, v, w, Tc):
    Nc, Kk, F = h.shape
    Fo = w.shape[1]
    G = Nc // Tc
    outs = [jax.ShapeDtypeStruct((Nc, Kk, Fo), _f32),
            jax.ShapeDtypeStruct((1, Fo), _f32),
            jax.ShapeDtypeStruct((1, Fo), _f32)]
    return pl.pallas_call(
        _bnmm_kernel,
        grid=(G,),
        in_specs=[
            pl.BlockSpec((Tc, Kk, F), lambda i: (i, 0, 0)),
            pl.BlockSpec((1, F), lambda i: (0, 0)),
            pl.BlockSpec((1, F), lambda i: (0, 0)),
            pl.BlockSpec((1, 1), lambda i: (0, 0)),
            pl.BlockSpec((Tc, Kk), lambda i: (i, 0)),
            pl.BlockSpec((F, Fo), lambda i: (0, 0)),
        ],
        out_specs=[
            pl.BlockSpec((Tc, Kk, Fo), lambda i: (i, 0, 0)),
            pl.BlockSpec((1, Fo), lambda i: (0, 0)),
            pl.BlockSpec((1, Fo), lambda i: (0, 0)),
        ],
        out_shape=outs,
    )(h, ss, sq, dn, v, w)


def _maxk_kernel(h_ref, ss_ref, sq_ref, dn_ref, v_ref, o_ref):
    den = dn_ref[0, 0]
    mu = ss_ref[...] / den
    var = sq_ref[...] / den - mu * mu
    F = mu.shape[1]
    x = (h_ref[...] - mu.reshape(1, 1, F)) / jnp.sqrt(var + _EPS).reshape(1, 1, F)
    x = jnp.maximum(x, 0.0)
    x = jnp.where(v_ref[...][..., None] > 0, x, _NEG)
    o_ref[...] = jnp.max(x, axis=1)


def _run_maxk(h3d, ss, sq, dn, v, Tc):
    Nc, Kk, F = h3d.shape
    G = Nc // Tc
    return pl.pallas_call(
        _maxk_kernel,
        grid=(G,),
        in_specs=[
            pl.BlockSpec((Tc, Kk, F), lambda i: (i, 0, 0)),
            pl.BlockSpec((1, F), lambda i: (0, 0)),
            pl.BlockSpec((1, F), lambda i: (0, 0)),
            pl.BlockSpec((1, 1), lambda i: (0, 0)),
            pl.BlockSpec((Tc, Kk), lambda i: (i, 0)),
        ],
        out_specs=pl.BlockSpec((Tc, F), lambda i: (i, 0)),
        out_shape=jax.ShapeDtypeStruct((Nc, F), _f32),
    )(h3d, ss, sq, dn, v)


def _s2l1_kernel(x1_ref, sc_ref, w_ref,
                 o_ref, ss_ref, sq_ref, dn_ref):
    x1 = x1_ref[0]
    w = w_ref[...]
    NR = _M2 * _K
    CH = 2048
    ps = jnp.zeros((1, 128), _f32)
    pq = jnp.zeros((1, 128), _f32)
    pd = jnp.zeros((1, 1), _f32)
    for c in range(NR // CH):
        sl = slice(c * CH, (c + 1) * CH)
        blk = sc_ref[0, sl, :]
        idx = blk[:, 0:1].astype(_i32)
        iota = jax.lax.broadcasted_iota(_i32, (CH, _M1), 1)
        oh = (iota == idx).astype(_f32)
        xj = jnp.dot(oh, x1, precision=jax.lax.Precision.HIGHEST,
                     preferred_element_type=_f32)
        msg = jnp.concatenate([xj, blk[:, 1:4]], axis=1)
        h = jnp.dot(msg, w, preferred_element_type=_f32)
        o_ref[0, sl, :] = h
        vv = blk[:, 4:5]
        hm = h * vv
        ps = ps + jnp.sum(hm, axis=0, keepdims=True)
        pq = pq + jnp.sum(h * hm, axis=0, keepdims=True)
        pd = pd + jnp.sum(vv, axis=0, keepdims=True)

    @pl.when(pl.program_id(0) == 0)
    def _init():
        ss_ref[...] = jnp.zeros_like(ss_ref)
        sq_ref[...] = jnp.zeros_like(sq_ref)
        dn_ref[...] = jnp.zeros_like(dn_ref)

    ss_ref[...] += ps
    sq_ref[...] += pq
    dn_ref[...] += pd


def _run_s2l1(x1b, scal, w2a):
    NR = _M2 * _K
    outs = [jax.ShapeDtypeStruct((_B, NR, 128), _f32),
            jax.ShapeDtypeStruct((1, 128), _f32),
            jax.ShapeDtypeStruct((1, 128), _f32),
            jax.ShapeDtypeStruct((1, 1), _f32)]
    return pl.pallas_call(
        _s2l1_kernel,
        grid=(_B,),
        in_specs=[
            pl.BlockSpec((1, _M1, 128), lambda b: (b, 0, 0)),
            pl.BlockSpec((1, NR, 5), lambda b: (b, 0, 0)),
            pl.BlockSpec((131, 128), lambda b: (0, 0)),
        ],
        out_specs=[
            pl.BlockSpec((1, NR, 128), lambda b: (b, 0, 0)),
            pl.BlockSpec((1, 128), lambda b: (0, 0)),
            pl.BlockSpec((1, 128), lambda b: (0, 0)),
            pl.BlockSpec((1, 1), lambda b: (0, 0)),
        ],
        out_shape=outs,
    )(x1b, scal, w2a)


def _knn3_gather(xd, yd, zd, xs, ys, zs, feat, M, N):
    dx = xd - xs
    dy = yd - ys
    dz = zd - zs
    d2 = (dx * dx + dy * dy) + dz * dz
    iota = jax.lax.broadcasted_iota(_i32, (M, N), 1)
    acc = jnp.zeros((M, feat.shape[1]), _f32)
    ws = jnp.zeros((M, 1), _f32)
    for _ in range(3):
        mn = jnp.min(d2, axis=1, keepdims=True)
        cand = jnp.where(d2 == mn, iota, N)
        idx = jnp.min(cand, axis=1, keepdims=True).astype(_i32)
        ohm = iota == idx
        oh = ohm.astype(_f32)
        w = 1.0 / jnp.maximum(mn, 1e-16)
        acc = acc + w * jnp.dot(oh, feat, precision=jax.lax.Precision.HIGHEST,
                                preferred_element_type=_f32)
        ws = ws + w
        d2 = jnp.where(ohm, jnp.inf, d2)
    return acc / ws


def _interp_f2_kernel(p1_ref, p2t_ref, x2_ref, x1_ref,
                      wf_ref, wb_ref, wc1_ref, wc2_ref, bc2_ref,
                      up_ref, lg_ref, h_ref):
    for b in range(_B):
        xd = p1_ref[b, :, 0:1]
        yd = p1_ref[b, :, 1:2]
        zd = p1_ref[b, :, 2:3]
        xs = p2t_ref[0, b:b + 1, :]
        ys = p2t_ref[1, b:b + 1, :]
        zs = p2t_ref[2, b:b + 1, :]
        xi = _knn3_gather(xd, yd, zd, xs, ys, zs, x2_ref[b], _M1, _M2)
        hcat = jnp.concatenate([xi, x1_ref[b]], axis=1)
        hb = jnp.dot(hcat, wf_ref[...], preferred_element_type=_f32)
        h_ref[b * _M1:(b + 1) * _M1, :] = hb
    n = float(_B * _M1)
    h = h_ref[...]
    mu = jnp.sum(h, axis=0, keepdims=True) / n
    var = jnp.sum((h - mu) ** 2, axis=0, keepdims=True) / n
    x = jnp.maximum((h - mu) / jnp.sqrt(var + _EPS), 0.0)
    h2 = jnp.dot(x, wb_ref[...], preferred_element_type=_f32)
    mu2 = jnp.sum(h2, axis=0, keepdims=True) / n
    var2 = jnp.sum((h2 - mu2) ** 2, axis=0, keepdims=True) / n
    up_ref[...] = jnp.maximum((h2 - mu2) / jnp.sqrt(var2 + _EPS), 0.0)
    g = jnp.max(x2_ref[...], axis=1)
    gh = jnp.dot(g, wc1_ref[...], preferred_element_type=_f32)
    mug = jnp.sum(gh, axis=0, keepdims=True) / float(_B)
    varg = jnp.sum((gh - mug) ** 2, axis=0, keepdims=True) / float(_B)
    ch = jnp.maximum((gh - mug) / jnp.sqrt(varg + _EPS), 0.0)
    lg_ref[...] = jnp.dot(ch, wc2_ref[...], preferred_element_type=_f32) + \
        bc2_ref[...]


def _run_interp_f2(p1r, p2t, x2b, x1b, wf2a, wb, wc1, wc2, bc2):
    outs = [jax.ShapeDtypeStruct((_B * _M1, 128), _f32),
            jax.ShapeDtypeStruct((_B, 40), _f32)]
    return pl.pallas_call(
        _interp_f2_kernel,
        out_shape=outs,
        scratch_shapes=[pltpu.VMEM((_B * _M1, 128), _f32)],
    )(p1r, p2t, x2b, x1b, wf2a, wb, wc1, wc2, bc2)


def _interp2_kernel(p0_ref, p1t_ref, up_ref,
                    wa_ref, wb_ref, wr1_ref, wr2_ref, br2_ref,
                    rec_ref, xi_ref):
    for b in range(_B):
        xd = p0_ref[b, :, 0:1]
        yd = p0_ref[b, :, 1:2]
        zd = p0_ref[b, :, 2:3]
        xs = p1t_ref[0, b:b + 1, :]
        ys = p1t_ref[1, b:b + 1, :]
        zs = p1t_ref[2, b:b + 1, :]
        xi = _knn3_gather(xd, yd, zd, xs, ys, zs, up_ref[b], _P, _M1)
        xi_ref[b * _P:(b + 1) * _P, :] = xi
    n = float(_B * _P)
    h = jnp.dot(xi_ref[...], wa_ref[...], preferred_element_type=_f32)
    mu = jnp.sum(h, axis=0, keepdims=True) / n
    var = jnp.sum((h - mu) ** 2, axis=0, keepdims=True) / n
    x = jnp.maximum((h - mu) / jnp.sqrt(var + _EPS), 0.0)
    h2 = jnp.dot(x, wb_ref[...], preferred_element_type=_f32)
    mu2 = jnp.sum(h2, axis=0, keepdims=True) / n
    var2 = jnp.sum((h2 - mu2) ** 2, axis=0, keepdims=True) / n
    x0 = jnp.maximum((h2 - mu2) / jnp.sqrt(var2 + _EPS), 0.0)
    h3 = jnp.dot(x0, wr1_ref[...], preferred_element_type=_f32)
    mu3 = jnp.sum(h3, axis=0, keepdims=True) / n
    var3 = jnp.sum((h3 - mu3) ** 2, axis=0, keepdims=True) / n
    rh = jnp.maximum((h3 - mu3) / jnp.sqrt(var3 + _EPS), 0.0)
    rec_ref[...] = jnp.dot(rh, wr2_ref[...], preferred_element_type=_f32) + \
        br2_ref[...]


def _run_interp2(p0r, p1t, upb, wa, wb, wr1, wr2, br2):
    return pl.pallas_call(
        _interp2_kernel,
        out_shape=jax.ShapeDtypeStruct((_B * _P, 3), _f32),
        scratch_shapes=[pltpu.VMEM((_B * _P, 128), _f32)],
    )(p0r, p1t, upb, wa, wb, wr1, wr2, br2)


def kernel(pos, batch, W1a, W1b, W1c, W2a, W2b, W2c, Wf2a, Wf2b,
           Wf1a, Wf1b, Wr1, Wr2, br2, Wc1, Wc2, bc2):
    posb = pos.reshape(_B, _P, 3)
    pos_t = posb.transpose(2, 0, 1)

    px1, py1, pz1, px2, py2, pz2 = _run_fps(pos_t)
    pos1_t = jnp.stack([px1, py1, pz1])
    pos1_r = jnp.stack([px1, py1, pz1], axis=-1)
    pos2_t = jnp.stack([px2, py2, pz2])
    pos2_r = jnp.stack([px2, py2, pz2], axis=-1)
    summary_pos = pos2_r.reshape(_B * _M2, 3)

    pos_4 = pos_t.transpose(1, 0, 2)[:, :, None, :]
    pos1_4 = pos1_t.transpose(1, 0, 2)[:, :, None, :]

    Nc1 = _B * _M1
    rx1, ry1, rz1, va1, _ = _run_radius(pos_4, pos1_r, _M1, _P, _R1SQ)
    va1c = va1.reshape(Nc1, _K)
    h1, s1s, s1q, s1d = _run_l1(
        rx1.reshape(Nc1, _K), ry1.reshape(Nc1, _K), rz1.reshape(Nc1, _K),
        va1c, W1a)
    h2, s2s, s2q = _run_bnmm(h1, s1s, s1q, s1d, va1c, W1b, Tc=256)
    h3, s3s, s3q = _run_bnmm(h2, s2s, s2q, s1d, va1c, W1c, Tc=256)
    x1 = _run_maxk(h3, s3s, s3q, s1d, va1c, Tc=128)
    x1b = x1.reshape(_B, _M1, 128)

    Nc2 = _B * _M2
    NR = _M2 * _K
    rx2, ry2, rz2, va2, ix2 = _run_radius(pos1_4, pos2_r, _M2, _M1, _R2SQ)
    scal = jnp.concatenate(
        [ix2.astype(_f32).reshape(_B, NR, 1), rx2.reshape(_B, NR, 1),
         ry2.reshape(_B, NR, 1), rz2.reshape(_B, NR, 1),
         va2.reshape(_B, NR, 1)], axis=-1)
    va2c = va2.reshape(Nc2, _K)
    g1, t1s, t1q, t1d = _run_s2l1(x1b, scal, W2a)
    g13 = g1.reshape(Nc2, _K, 128)
    g2, t2s, t2q = _run_bnmm(g13, t1s, t1q, t1d, va2c, W2b, Tc=128)
    g3, t3s, t3q = _run_bnmm(g2, t2s, t2q, t1d, va2c, W2c, Tc=64)
    x2 = _run_maxk(g3, t3s, t3q, t1d, va2c, Tc=64)
    x2b = x2.reshape(_B, _M2, 256)

    x1up, logits = _run_interp_f2(
        pos1_r, pos2_t, x2b, x1b,
        Wf2a, Wf2b, Wc1, Wc2, bc2.reshape(1, 40))
    recon = _run_interp2(
        posb, pos1_t, x1up.reshape(_B, _M1, 128),
        Wf1a, Wf1b, Wr1, Wr2, br2.reshape(1, 3))

    return summary_pos, recon, logits
